# baseline (speedup 1.0000x reference)
import functools

import jax
import jax.numpy as jnp
from jax import lax
from jax.experimental import pallas as pl
from jax.experimental.pallas import tpu as pltpu
from jax.experimental.pallas import tpu_sc as plsc

_INTERPRET = False


def _s2d(x):
    n, h, w, c = x.shape
    x = x.reshape(n, h // 2, 2, w // 2, 2, c)
    x = jnp.transpose(x, (0, 1, 3, 2, 4, 5))
    return x.reshape(n, h // 2, w // 2, 4 * c)


def _s2d_inv(x):
    n, h, w, c4 = x.shape
    c = c4 // 4
    x = x.reshape(n, h, w, 2, 2, c)
    x = jnp.transpose(x, (0, 1, 3, 2, 4, 5))
    return x.reshape(n, 2 * h, 2 * w, c)


_S2_TAP = ((-1, 1), (0, 0), (0, 1), (1, 0))


def _s2w(w):
    o, i = w.shape[0], w.shape[1]
    out = jnp.zeros((3, 3, 4 * i, o), jnp.float32)
    for ky in range(4):
        dy, py = _S2_TAP[ky]
        for kx in range(4):
            dx, px = _S2_TAP[kx]
            g = py * 2 + px
            out = out.at[dy + 1, dx + 1, g * i:(g + 1) * i, :].set(w[:, :, ky, kx].T)
    return out


_CT_TAP = {0: ((1, 0), (3, -1)), 1: ((0, 1), (2, 0))}


def _ctw(w):
    i, o = w.shape[0], w.shape[1]
    out = jnp.zeros((3, 3, i, 4 * o), jnp.float32)
    for py in (0, 1):
        for ky, dy in _CT_TAP[py]:
            for px in (0, 1):
                for kx, dx in _CT_TAP[px]:
                    g = py * 2 + px
                    out = out.at[dy + 1, dx + 1, :, g * o:(g + 1) * o].set(w[:, :, ky, kx])
    return out


def _w3x3(w):
    return jnp.transpose(w, (2, 3, 1, 0))


def _conv(x, w, b, relu_in=False, relu_out=False, pw=None, pb=None,
          out_dtype=jnp.bfloat16, nchunk=1):
    n, h, wd, c = x.shape
    co = w.shape[-1]
    cf = pw.shape[-1] if pw is not None else co
    w3 = w.reshape(3, 3 * c, co)
    bb = b.reshape(1, co)

    ins = [x, w3, bb]
    in_specs = [
        pl.BlockSpec((1, h, wd, c), lambda ni: (ni, 0, 0, 0)),
        pl.BlockSpec((3, 3 * c, co), lambda ni: (0, 0, 0)),
        pl.BlockSpec((1, co), lambda ni: (0, 0)),
    ]
    if pw is not None:
        ins += [pw, pb.reshape(1, cf)]
        in_specs += [
            pl.BlockSpec((co, cf), lambda ni: (0, 0)),
            pl.BlockSpec((1, cf), lambda ni: (0, 0)),
        ]
    rows = h // nchunk

    def body(*refs):
        if pw is None:
            x_ref, w_ref, b_ref, o_ref = refs
        else:
            x_ref, w_ref, b_ref, pw_ref, pb_ref, o_ref = refs
        xb = x_ref[0]
        if relu_in:
            xb = jnp.maximum(xb, 0.0)
        xb = xb.astype(jnp.bfloat16)
        zr = jnp.zeros((1, wd, 3 * c), jnp.bfloat16)
        wl = [w_ref[dy].astype(jnp.bfloat16) for dy in range(3)]
        for k in range(nchunk):
            lo = max(k * rows - 1, 0)
            hi = min(k * rows + rows + 1, h)
            ctr = xb[lo:hi]
            zc = jnp.zeros((hi - lo, 1, c), jnp.bfloat16)
            lft = jnp.concatenate([zc, ctr[:, :wd - 1]], axis=1)
            rgt = jnp.concatenate([ctr[:, 1:], zc], axis=1)
            pc = jnp.concatenate([lft, ctr, rgt], axis=-1)
            if k == 0:
                pc = jnp.concatenate([zr, pc], axis=0)
            if k == nchunk - 1:
                pc = jnp.concatenate([pc, zr], axis=0)
            acc = None
            for dy in range(3):
                seg = pc[dy:dy + rows]
                t = jnp.dot(seg.reshape(rows * wd, 3 * c), wl[dy],
                            preferred_element_type=jnp.float32)
                acc = t if acc is None else acc + t
            acc = acc + b_ref[:]
            if relu_out or pw is not None:
                acc = jnp.maximum(acc, 0.0)
            if pw is not None:
                acc = jnp.dot(acc.astype(jnp.bfloat16),
                              pw_ref[:].astype(jnp.bfloat16),
                              preferred_element_type=jnp.float32) + pb_ref[:]
            o_ref[0, pl.ds(k * rows, rows)] = acc.reshape(
                rows, wd, cf).astype(out_dtype)

    return pl.pallas_call(
        body,
        grid=(n,),
        in_specs=in_specs,
        out_specs=pl.BlockSpec((1, h, wd, cf), lambda ni: (ni, 0, 0, 0)),
        out_shape=jax.ShapeDtypeStruct((n, h, wd, cf), out_dtype),
        interpret=_INTERPRET,
    )(*ins)


def _conv_val(xb, w3_ref, b_ref, relu_in=False, relu_out=False,
              pw_ref=None, pb_ref=None):
    h, wd, c = xb.shape
    if relu_in:
        xb = jnp.maximum(xb, 0.0)
    xb = xb.astype(jnp.bfloat16)
    zc = jnp.zeros((h, 1, c), jnp.bfloat16)
    lft = jnp.concatenate([zc, xb[:, :wd - 1]], axis=1)
    rgt = jnp.concatenate([xb[:, 1:], zc], axis=1)
    pc = jnp.concatenate([lft, xb, rgt], axis=-1)
    zr = jnp.zeros((1, wd, 3 * c), jnp.bfloat16)
    pc = jnp.concatenate([zr, pc, zr], axis=0)
    acc = None
    for dy in range(3):
        t = jnp.dot(pc[dy:dy + h].reshape(h * wd, 3 * c),
                    w3_ref[dy].astype(jnp.bfloat16),
                    preferred_element_type=jnp.float32)
        acc = t if acc is None else acc + t
    acc = acc + b_ref[:]
    if relu_out or pw_ref is not None:
        acc = jnp.maximum(acc, 0.0)
    if pw_ref is not None:
        acc = jnp.dot(acc.astype(jnp.bfloat16),
                      pw_ref[:].astype(jnp.bfloat16),
                      preferred_element_type=jnp.float32) + pb_ref[:]
    cf = acc.shape[-1]
    return acc.reshape(h, wd, cf).astype(jnp.bfloat16)


def _chain(x, stages):
    n, h, wd, c = x.shape
    ins = [x]
    in_specs = [pl.BlockSpec((1, h, wd, c), lambda ni: (ni, 0, 0, 0))]
    counts = []
    for st in stages:
        c_in = st["w"].shape[2]
        co = st["w"].shape[3]
        args = [st["w"].reshape(3, 3 * c_in, co), st["b"].reshape(1, co)]
        specs = [
            pl.BlockSpec((3, 3 * c_in, co), lambda ni: (0, 0, 0)),
            pl.BlockSpec((1, co), lambda ni: (0, 0)),
        ]
        if st.get("pw") is not None:
            cf = st["pw"].shape[1]
            args += [st["pw"], st["pb"].reshape(1, cf)]
            specs += [
                pl.BlockSpec((co, cf), lambda ni: (0, 0)),
                pl.BlockSpec((1, cf), lambda ni: (0, 0)),
            ]
        ins += args
        in_specs += specs
        counts.append(len(args))
    cf_out = (stages[-1]["pw"].shape[1] if stages[-1].get("pw") is not None
              else stages[-1]["w"].shape[3])

    def body(*refs):
        x_ref, o_ref = refs[0], refs[-1]
        wrefs = refs[1:-1]
        v = x_ref[0]
        pos = 0
        for st, cnt in zip(stages, counts):
            r = wrefs[pos:pos + cnt]
            pos += cnt
            pw_ref, pb_ref = (r[2], r[3]) if cnt == 4 else (None, None)
            v = _conv_val(v, r[0], r[1], relu_in=st.get("relu_in", False),
                          relu_out=st.get("relu_out", False),
                          pw_ref=pw_ref, pb_ref=pb_ref)
        o_ref[0] = v

    return pl.pallas_call(
        body,
        grid=(n,),
        in_specs=in_specs,
        out_specs=pl.BlockSpec((1, h, wd, cf_out), lambda ni: (ni, 0, 0, 0)),
        out_shape=jax.ShapeDtypeStruct((n, h, wd, cf_out), jnp.bfloat16),
        interpret=_INTERPRET,
    )(*ins)


def _im2col3(x):
    n, h, w, c = x.shape
    xp = jnp.pad(x, ((0, 0), (1, 1), (1, 1), (0, 0)))
    cols = [xp[:, dy:dy + h, dx:dx + w, :]
            for dy in range(3) for dx in range(3)]
    return jnp.concatenate(cols, axis=-1).reshape(n * h * w, 9 * c)


def _mm(xcol, w2, b, relu_out=False, out_dtype=jnp.bfloat16, mblk=4608):
    m, k = xcol.shape
    nn = w2.shape[1]

    def body(x_ref, w_ref, b_ref, o_ref):
        t = jnp.dot(x_ref[:].astype(jnp.bfloat16),
                    w_ref[:].astype(jnp.bfloat16),
                    preferred_element_type=jnp.float32) + b_ref[:]
        if relu_out:
            t = jnp.maximum(t, 0.0)
        o_ref[:] = t.astype(out_dtype)

    return pl.pallas_call(
        body,
        grid=(m // mblk,),
        in_specs=[
            pl.BlockSpec((mblk, k), lambda i: (i, 0)),
            pl.BlockSpec((k, nn), lambda i: (0, 0)),
            pl.BlockSpec((1, nn), lambda i: (0, 0)),
        ],
        out_specs=pl.BlockSpec((mblk, nn), lambda i: (i, 0)),
        out_shape=jax.ShapeDtypeStruct((m, nn), out_dtype),
        interpret=_INTERPRET,
    )(xcol, w2, b.reshape(1, nn))


def _vq(zf, embed, m):
    bsz, d = zf.shape
    e = embed.shape[0]
    nb = bsz // m
    et = embed.T

    def body(z_ref, et_ref, idx_ref, ls_ref):
        z = z_ref[:].astype(jnp.float32)
        ze = jnp.dot(z, et_ref[:], preferred_element_type=jnp.float32)
        z2 = jnp.sum(z * z, axis=1, keepdims=True)
        e2 = jnp.sum(et_ref[:] * et_ref[:], axis=0, keepdims=True)
        dist = z2 - 2.0 * ze + e2
        dmin = jnp.min(dist, axis=1, keepdims=True)
        ii = lax.broadcasted_iota(jnp.int32, dist.shape, 1)
        idxv = jnp.min(jnp.where(dist == dmin, ii, e), axis=1, keepdims=True)
        idx_ref[:] = idxv.astype(jnp.int32)
        s = jnp.sum(dmin).reshape(1, 1)

        @pl.when(pl.program_id(0) == 0)
        def _():
            ls_ref[:] = s

        @pl.when(pl.program_id(0) != 0)
        def _():
            ls_ref[:] = ls_ref[:] + s

    idx, ls = pl.pallas_call(
        body,
        grid=(nb,),
        in_specs=[
            pl.BlockSpec((m, d), lambda i: (i, 0)),
            pl.BlockSpec((d, e), lambda i: (0, 0)),
        ],
        out_specs=[
            pl.BlockSpec((m, 1), lambda i: (i, 0)),
            pl.BlockSpec((1, 1), lambda i: (0, 0)),
        ],
        out_shape=[
            jax.ShapeDtypeStruct((bsz, 1), jnp.int32),
            jax.ShapeDtypeStruct((1, 1), jnp.float32),
        ],
        interpret=_INTERPRET,
    )(zf, et)
    return idx.reshape(bsz), ls[0, 0]


def _gather_zq(embed, idx):
    e, d = embed.shape
    bsz = idx.shape[0]
    info = plsc.get_sparse_core_info()
    nc, ns = info.num_cores, info.num_subcores
    nw = nc * ns
    bpw = bsz // nw
    mesh = plsc.VectorSubcoreMesh(core_axis_name="c", subcore_axis_name="s")

    @functools.partial(
        pl.kernel,
        mesh=mesh,
        out_type=jax.ShapeDtypeStruct((bsz, d), jnp.float32),
        scratch_types=[
            pltpu.VMEM((bpw,), jnp.int32),
            pltpu.VMEM((bpw, d), jnp.float32),
            pltpu.VMEM((e, d), jnp.float32),
            pltpu.SemaphoreType.DMA,
        ],
        compiler_params=pltpu.CompilerParams(
            use_tc_tiling_on_sc=False, needs_layout_passes=False),
    )
    def k(table_hbm, idx_hbm, out_hbm, idx_v, rows_v, tbl_v, sem):
        wid = lax.axis_index("s") * nc + lax.axis_index("c")
        base = wid * bpw
        pltpu.sync_copy(table_hbm, tbl_v)
        pltpu.sync_copy(idx_hbm.at[pl.ds(base, bpw)], idx_v)
        lanes = 16
        ngroup = bpw // lanes

        def group(g, carry):
            iv = idx_v[pl.ds(g * lanes, lanes)]
            riota = g * lanes + lax.iota(jnp.int32, lanes)
            for c in range(d):
                cc = jnp.full((lanes,), c, jnp.int32)
                v = plsc.load_gather(tbl_v, [iv, cc])
                plsc.store_scatter(rows_v, [riota, cc], v)
            return carry

        lax.fori_loop(0, ngroup, group, 0)
        pltpu.sync_copy(rows_v, out_hbm.at[pl.ds(base, bpw)])

    return k(embed, idx)


def kernel(x, enc_w1, enc_b1, enc_w2, enc_b2, enc_w3, enc_b3, enc_res_w1,
           enc_res_b1, enc_res_w2, enc_res_b2, embed, dec_w1, dec_b1,
           dec_res_w1, dec_res_b1, dec_res_w2, dec_res_b2, dec_w2, dec_b2,
           dec_w3, dec_b3):
    n = x.shape[0]
    a = jnp.transpose(x, (0, 2, 3, 1))
    a = _s2d(a).astype(jnp.bfloat16)
    a = _mm(_im2col3(a), _s2w(enc_w1).reshape(36, -1), enc_b1,
            relu_out=True)
    a = _s2d(a.reshape(n, 192, 192, -1))
    enc_stages = [
        {"w": _s2w(enc_w2), "b": enc_b2, "relu_out": True},
        {"w": _w3x3(enc_w3), "b": enc_b3},
    ]
    for i in range(enc_res_w1.shape[0]):
        enc_stages.append({"w": _w3x3(enc_res_w1[i]), "b": enc_res_b1[i],
                           "relu_in": True,
                           "pw": enc_res_w2[i][:, :, 0, 0].T,
                           "pb": enc_res_b2[i]})
    a = _chain(a, enc_stages)

    zf = a.reshape(-1, embed.shape[1])
    idx, lsum = _vq(zf, embed, m=1024)
    loss = 1.25 * lsum / zf.size
    zq = _gather_zq(embed, idx)

    t = zq.reshape(n, 96, 96, embed.shape[1])
    dec_stages = [
        {"w": jnp.transpose(jnp.flip(dec_w1, (2, 3)), (2, 3, 0, 1)),
         "b": dec_b1},
    ]
    for i in range(dec_res_w1.shape[0]):
        dec_stages.append({"w": _w3x3(dec_res_w1[i]), "b": dec_res_b1[i],
                           "relu_in": True,
                           "pw": dec_res_w2[i][:, :, 0, 0].T,
                           "pb": dec_res_b2[i]})
    dec_stages.append({"w": _ctw(dec_w2), "b": jnp.tile(dec_b2, 4),
                       "relu_out": True})
    t = _chain(t, dec_stages)
    t = _s2d_inv(t)
    t = _mm(_im2col3(t), _ctw(dec_w3).reshape(576, -1),
            jnp.tile(dec_b3, 4), out_dtype=jnp.float32)
    xr = _s2d_inv(t.reshape(n, 192, 192, 4))
    return jnp.transpose(xr, (0, 3, 1, 2)), loss


# speedup vs baseline: 1.6645x; 1.6645x over previous
import functools

import jax
import jax.numpy as jnp
from jax import lax
from jax.experimental import pallas as pl
from jax.experimental.pallas import tpu as pltpu
from jax.experimental.pallas import tpu_sc as plsc

_INTERPRET = False


def _s2d(x):
    n, h, w, c = x.shape
    x = x.reshape(n, h // 2, 2, w // 2, 2, c)
    x = jnp.transpose(x, (0, 1, 3, 2, 4, 5))
    return x.reshape(n, h // 2, w // 2, 4 * c)


def _s2d_inv(x):
    n, h, w, c4 = x.shape
    c = c4 // 4
    x = x.reshape(n, h, w, 2, 2, c)
    x = jnp.transpose(x, (0, 1, 3, 2, 4, 5))
    return x.reshape(n, 2 * h, 2 * w, c)


_S2_TAP = ((-1, 1), (0, 0), (0, 1), (1, 0))


def _s2w(w):
    o, i = w.shape[0], w.shape[1]
    out = jnp.zeros((3, 3, 4 * i, o), jnp.float32)
    for ky in range(4):
        dy, py = _S2_TAP[ky]
        for kx in range(4):
            dx, px = _S2_TAP[kx]
            g = py * 2 + px
            out = out.at[dy + 1, dx + 1, g * i:(g + 1) * i, :].set(w[:, :, ky, kx].T)
    return out


_CT_TAP = {0: ((1, 0), (3, -1)), 1: ((0, 1), (2, 0))}


def _ctw(w):
    i, o = w.shape[0], w.shape[1]
    out = jnp.zeros((3, 3, i, 4 * o), jnp.float32)
    for py in (0, 1):
        for ky, dy in _CT_TAP[py]:
            for px in (0, 1):
                for kx, dx in _CT_TAP[px]:
                    g = py * 2 + px
                    out = out.at[dy + 1, dx + 1, :, g * o:(g + 1) * o].set(w[:, :, ky, kx])
    return out


def _w3x3(w):
    return jnp.transpose(w, (2, 3, 1, 0))


def _lift(w):
    c, g = w.shape[2], w.shape[3]
    out = jnp.zeros((3, 3, 4 * c, 4 * g), jnp.float32)
    for qy in (0, 1):
        for dy in (-1, 0, 1):
            sy = qy + dy
            ey, py = sy // 2, sy % 2
            for qx in (0, 1):
                for dx in (-1, 0, 1):
                    sx = qx + dx
                    ex, px = sx // 2, sx % 2
                    gi = py * 2 + px
                    go = qy * 2 + qx
                    out = out.at[ey + 1, ex + 1, gi * c:(gi + 1) * c,
                                 go * g:(go + 1) * g].set(w[dy + 1, dx + 1])
    return out


def _conv(x, w, b, relu_in=False, relu_out=False, pw=None, pb=None,
          out_dtype=jnp.bfloat16, nchunk=1):
    n, h, wd, c = x.shape
    co = w.shape[-1]
    cf = pw.shape[-1] if pw is not None else co
    w3 = w.reshape(3, 3 * c, co)
    bb = b.reshape(1, co)

    ins = [x, w3, bb]
    in_specs = [
        pl.BlockSpec((1, h, wd, c), lambda ni: (ni, 0, 0, 0)),
        pl.BlockSpec((3, 3 * c, co), lambda ni: (0, 0, 0)),
        pl.BlockSpec((1, co), lambda ni: (0, 0)),
    ]
    if pw is not None:
        ins += [pw, pb.reshape(1, cf)]
        in_specs += [
            pl.BlockSpec((co, cf), lambda ni: (0, 0)),
            pl.BlockSpec((1, cf), lambda ni: (0, 0)),
        ]
    rows = h // nchunk

    def body(*refs):
        if pw is None:
            x_ref, w_ref, b_ref, o_ref = refs
        else:
            x_ref, w_ref, b_ref, pw_ref, pb_ref, o_ref = refs
        xb = x_ref[0]
        if relu_in:
            xb = jnp.maximum(xb, 0.0)
        xb = xb.astype(jnp.bfloat16)
        zr = jnp.zeros((1, wd, 3 * c), jnp.bfloat16)
        wl = [w_ref[dy].astype(jnp.bfloat16) for dy in range(3)]
        for k in range(nchunk):
            lo = max(k * rows - 1, 0)
            hi = min(k * rows + rows + 1, h)
            ctr = xb[lo:hi]
            zc = jnp.zeros((hi - lo, 1, c), jnp.bfloat16)
            lft = jnp.concatenate([zc, ctr[:, :wd - 1]], axis=1)
            rgt = jnp.concatenate([ctr[:, 1:], zc], axis=1)
            pc = jnp.concatenate([lft, ctr, rgt], axis=-1)
            if k == 0:
                pc = jnp.concatenate([zr, pc], axis=0)
            if k == nchunk - 1:
                pc = jnp.concatenate([pc, zr], axis=0)
            acc = None
            for dy in range(3):
                seg = pc[dy:dy + rows]
                t = jnp.dot(seg.reshape(rows * wd, 3 * c), wl[dy],
                            preferred_element_type=jnp.float32)
                acc = t if acc is None else acc + t
            acc = acc + b_ref[:]
            if relu_out or pw is not None:
                acc = jnp.maximum(acc, 0.0)
            if pw is not None:
                acc = jnp.dot(acc.astype(jnp.bfloat16),
                              pw_ref[:].astype(jnp.bfloat16),
                              preferred_element_type=jnp.float32) + pb_ref[:]
            o_ref[0, pl.ds(k * rows, rows)] = acc.reshape(
                rows, wd, cf).astype(out_dtype)

    return pl.pallas_call(
        body,
        grid=(n,),
        in_specs=in_specs,
        out_specs=pl.BlockSpec((1, h, wd, cf), lambda ni: (ni, 0, 0, 0)),
        out_shape=jax.ShapeDtypeStruct((n, h, wd, cf), out_dtype),
        interpret=_INTERPRET,
    )(*ins)


def _conv_val(xb, w3_ref, b_ref, relu_in=False, relu_out=False,
              pw_ref=None, pb_ref=None):
    h, wd, c = xb.shape
    if relu_in:
        xb = jnp.maximum(xb, 0.0)
    xb = xb.astype(jnp.bfloat16)
    zc = jnp.zeros((h, 1, c), jnp.bfloat16)
    lft = jnp.concatenate([zc, xb[:, :wd - 1]], axis=1)
    rgt = jnp.concatenate([xb[:, 1:], zc], axis=1)
    pc = jnp.concatenate([lft, xb, rgt], axis=-1)
    zr = jnp.zeros((1, wd, 3 * c), jnp.bfloat16)
    pc = jnp.concatenate([zr, pc, zr], axis=0)
    acc = None
    for dy in range(3):
        t = jnp.dot(pc[dy:dy + h].reshape(h * wd, 3 * c),
                    w3_ref[dy].astype(jnp.bfloat16),
                    preferred_element_type=jnp.float32)
        acc = t if acc is None else acc + t
    acc = acc + b_ref[:]
    if relu_out or pw_ref is not None:
        acc = jnp.maximum(acc, 0.0)
    if pw_ref is not None:
        acc = jnp.dot(acc.astype(jnp.bfloat16),
                      pw_ref[:].astype(jnp.bfloat16),
                      preferred_element_type=jnp.float32) + pb_ref[:]
    cf = acc.shape[-1]
    return acc.reshape(h, wd, cf).astype(jnp.bfloat16)


def _chain(x, stages, out_dtype=jnp.bfloat16):
    n, h, wd, c = x.shape
    ins = [x]
    in_specs = [pl.BlockSpec((1, h, wd, c), lambda ni: (ni, 0, 0, 0))]
    counts = []
    for st in stages:
        c_in = st["w"].shape[2]
        co = st["w"].shape[3]
        args = [st["w"].reshape(3, 3 * c_in, co), st["b"].reshape(1, co)]
        specs = [
            pl.BlockSpec((3, 3 * c_in, co), lambda ni: (0, 0, 0)),
            pl.BlockSpec((1, co), lambda ni: (0, 0)),
        ]
        if st.get("pw") is not None:
            cf = st["pw"].shape[1]
            args += [st["pw"], st["pb"].reshape(1, cf)]
            specs += [
                pl.BlockSpec((co, cf), lambda ni: (0, 0)),
                pl.BlockSpec((1, cf), lambda ni: (0, 0)),
            ]
        ins += args
        in_specs += specs
        counts.append(len(args))
    cf_out = (stages[-1]["pw"].shape[1] if stages[-1].get("pw") is not None
              else stages[-1]["w"].shape[3])

    def body(*refs):
        x_ref, o_ref = refs[0], refs[-1]
        wrefs = refs[1:-1]
        v = x_ref[0]
        pos = 0
        for st, cnt in zip(stages, counts):
            r = wrefs[pos:pos + cnt]
            pos += cnt
            pw_ref, pb_ref = (r[2], r[3]) if cnt == 4 else (None, None)
            v = _conv_val(v, r[0], r[1], relu_in=st.get("relu_in", False),
                          relu_out=st.get("relu_out", False),
                          pw_ref=pw_ref, pb_ref=pb_ref)
        o_ref[0] = v.astype(out_dtype)

    return pl.pallas_call(
        body,
        grid=(n,),
        in_specs=in_specs,
        out_specs=pl.BlockSpec((1, h, wd, cf_out), lambda ni: (ni, 0, 0, 0)),
        out_shape=jax.ShapeDtypeStruct((n, h, wd, cf_out), out_dtype),
        interpret=_INTERPRET,
    )(*ins)


def _im2col3(x):
    n, h, w, c = x.shape
    xp = jnp.pad(x, ((0, 0), (1, 1), (1, 1), (0, 0)))
    cols = [xp[:, dy:dy + h, dx:dx + w, :]
            for dy in range(3) for dx in range(3)]
    return jnp.concatenate(cols, axis=-1).reshape(n * h * w, 9 * c)


def _mm(xcol, w2, b, relu_out=False, out_dtype=jnp.bfloat16, mblk=4608):
    m, k = xcol.shape
    nn = w2.shape[1]

    def body(x_ref, w_ref, b_ref, o_ref):
        t = jnp.dot(x_ref[:].astype(jnp.bfloat16),
                    w_ref[:].astype(jnp.bfloat16),
                    preferred_element_type=jnp.float32) + b_ref[:]
        if relu_out:
            t = jnp.maximum(t, 0.0)
        o_ref[:] = t.astype(out_dtype)

    return pl.pallas_call(
        body,
        grid=(m // mblk,),
        in_specs=[
            pl.BlockSpec((mblk, k), lambda i: (i, 0)),
            pl.BlockSpec((k, nn), lambda i: (0, 0)),
            pl.BlockSpec((1, nn), lambda i: (0, 0)),
        ],
        out_specs=pl.BlockSpec((mblk, nn), lambda i: (i, 0)),
        out_shape=jax.ShapeDtypeStruct((m, nn), out_dtype),
        interpret=_INTERPRET,
    )(xcol, w2, b.reshape(1, nn))


def _vq(zf, embed, m):
    bsz, d = zf.shape
    e = embed.shape[0]
    nb = bsz // m
    et = embed.T

    def body(z_ref, et_ref, idx_ref, ls_ref):
        z = z_ref[:].astype(jnp.float32)
        ze = jnp.dot(z, et_ref[:], preferred_element_type=jnp.float32)
        z2 = jnp.sum(z * z, axis=1, keepdims=True)
        e2 = jnp.sum(et_ref[:] * et_ref[:], axis=0, keepdims=True)
        dist = z2 - 2.0 * ze + e2
        dmin = jnp.min(dist, axis=1, keepdims=True)
        ii = lax.broadcasted_iota(jnp.int32, dist.shape, 1)
        idxv = jnp.min(jnp.where(dist == dmin, ii, e), axis=1, keepdims=True)
        idx_ref[:] = idxv.astype(jnp.int32)
        s = jnp.sum(dmin).reshape(1, 1)

        @pl.when(pl.program_id(0) == 0)
        def _():
            ls_ref[:] = s

        @pl.when(pl.program_id(0) != 0)
        def _():
            ls_ref[:] = ls_ref[:] + s

    idx, ls = pl.pallas_call(
        body,
        grid=(nb,),
        in_specs=[
            pl.BlockSpec((m, d), lambda i: (i, 0)),
            pl.BlockSpec((d, e), lambda i: (0, 0)),
        ],
        out_specs=[
            pl.BlockSpec((m, 1), lambda i: (i, 0)),
            pl.BlockSpec((1, 1), lambda i: (0, 0)),
        ],
        out_shape=[
            jax.ShapeDtypeStruct((bsz, 1), jnp.int32),
            jax.ShapeDtypeStruct((1, 1), jnp.float32),
        ],
        interpret=_INTERPRET,
    )(zf, et)
    return idx.reshape(bsz), ls[0, 0]


def _gather_zq(embed, idx):
    e, d = embed.shape
    bsz = idx.shape[0]
    info = plsc.get_sparse_core_info()
    nc, ns = info.num_cores, info.num_subcores
    nw = nc * ns
    bpw = bsz // nw
    mesh = plsc.VectorSubcoreMesh(core_axis_name="c", subcore_axis_name="s")

    @functools.partial(
        pl.kernel,
        mesh=mesh,
        out_type=jax.ShapeDtypeStruct((bsz, d), jnp.float32),
        scratch_types=[
            pltpu.VMEM((bpw,), jnp.int32),
            pltpu.VMEM((bpw, d), jnp.float32),
            pltpu.VMEM((e, d), jnp.float32),
            pltpu.SemaphoreType.DMA,
        ],
        compiler_params=pltpu.CompilerParams(
            use_tc_tiling_on_sc=False, needs_layout_passes=False),
    )
    def k(table_hbm, idx_hbm, out_hbm, idx_v, rows_v, tbl_v, sem):
        wid = lax.axis_index("s") * nc + lax.axis_index("c")
        base = wid * bpw
        pltpu.sync_copy(table_hbm, tbl_v)
        pltpu.sync_copy(idx_hbm.at[pl.ds(base, bpw)], idx_v)
        lanes = 16
        ngroup = bpw // lanes

        def group(g, carry):
            iv = idx_v[pl.ds(g * lanes, lanes)]
            riota = g * lanes + lax.iota(jnp.int32, lanes)
            for c in range(d):
                cc = jnp.full((lanes,), c, jnp.int32)
                v = plsc.load_gather(tbl_v, [iv, cc])
                plsc.store_scatter(rows_v, [riota, cc], v)
            return carry

        lax.fori_loop(0, ngroup, group, 0)
        pltpu.sync_copy(rows_v, out_hbm.at[pl.ds(base, bpw)])

    return k(embed, idx)


def kernel(x, enc_w1, enc_b1, enc_w2, enc_b2, enc_w3, enc_b3, enc_res_w1,
           enc_res_b1, enc_res_w2, enc_res_b2, embed, dec_w1, dec_b1,
           dec_res_w1, dec_res_b1, dec_res_w2, dec_res_b2, dec_w2, dec_b2,
           dec_w3, dec_b3):
    n = x.shape[0]
    a = jnp.transpose(x, (0, 2, 3, 1))
    a = _s2d(_s2d(a)).astype(jnp.bfloat16)
    enc_stages = [
        {"w": _lift(_s2w(enc_w1)), "b": jnp.tile(enc_b1, 4),
         "relu_out": True},
        {"w": _s2w(enc_w2), "b": enc_b2, "relu_out": True},
        {"w": _w3x3(enc_w3), "b": enc_b3},
    ]
    for i in range(enc_res_w1.shape[0]):
        enc_stages.append({"w": _w3x3(enc_res_w1[i]), "b": enc_res_b1[i],
                           "relu_in": True,
                           "pw": enc_res_w2[i][:, :, 0, 0].T,
                           "pb": enc_res_b2[i]})
    a = _chain(a, enc_stages)

    zf = a.reshape(-1, embed.shape[1])
    idx, lsum = _vq(zf, embed, m=1024)
    loss = 1.25 * lsum / zf.size
    zq = _gather_zq(embed, idx)

    t = zq.reshape(n, 96, 96, embed.shape[1])
    dec_stages = [
        {"w": jnp.transpose(jnp.flip(dec_w1, (2, 3)), (2, 3, 0, 1)),
         "b": dec_b1},
    ]
    for i in range(dec_res_w1.shape[0]):
        dec_stages.append({"w": _w3x3(dec_res_w1[i]), "b": dec_res_b1[i],
                           "relu_in": True,
                           "pw": dec_res_w2[i][:, :, 0, 0].T,
                           "pb": dec_res_b2[i]})
    dec_stages.append({"w": _ctw(dec_w2), "b": jnp.tile(dec_b2, 4),
                       "relu_out": True})
    dec_stages.append({"w": _lift(_ctw(dec_w3)),
                       "b": jnp.tile(dec_b3, 16)})
    t = _chain(t, dec_stages, out_dtype=jnp.float32)
    xr = _s2d_inv(_s2d_inv(t))
    return jnp.transpose(xr, (0, 3, 1, 2)), loss
